# baseline (speedup 1.0000x reference)
import jax
import jax.numpy as jnp
from jax.experimental import pallas as pl
from jax.experimental.pallas import tpu as pltpu

HID = 768
NE = 32
EW = 2944
TOPK = 4
TD = 4
MAXNNZ = 512
ALPHA = 1.702
LIMIT = 7.0
OED = 2880

EB = 128
NB = EW // EB


def _router_body(x_ref, gw_ref, mask_ref, w_ref, slot_ref):
    x = x_ref[...]
    gw = gw_ref[...]
    bs = x.shape[0]
    f32 = jnp.float32
    i32 = jnp.int32

    logits = jax.lax.dot_general(x, gw, (((1,), (1,)), ((), ())),
                                 preferred_element_type=f32)
    m = jnp.max(logits, axis=1, keepdims=True)
    ex = jnp.exp(logits - m)
    rw = ex / jnp.sum(ex, axis=1, keepdims=True)

    iota_l = jax.lax.broadcasted_iota(i32, (bs, NE), 1)
    cur = rw
    srw_un = jnp.zeros_like(rw)
    vsum = jnp.zeros((bs, 1), f32)
    for _ in range(TOPK):
        mj = jnp.max(cur, axis=1, keepdims=True)
        ismax = cur == mj
        fidx = jnp.min(jnp.where(ismax, iota_l, NE * 2), axis=1, keepdims=True)
        oh = iota_l == fidx
        srw_un = srw_un + jnp.where(oh, mj, 0.0)
        vsum = vsum + mj
        cur = jnp.where(oh, -1.0, cur)
    srw = srw_un / vsum

    counts = jnp.sum((srw > 0.0).astype(f32), axis=0, keepdims=True)
    iota_e = jax.lax.broadcasted_iota(i32, (1, NE), 1)
    kv = counts.astype(i32) * NE + (NE - 1 - iota_e)
    e_list = []
    for _ in range(2 * TD):
        kmax = jnp.max(kv)
        e_j = jnp.min(jnp.where(kv == kmax, iota_e, NE * 2))
        e_list.append(e_j)
        kv = jnp.where(iota_e == e_j, -1, kv)
    mask_ref[...] = jnp.concatenate(
        [jnp.full((1, 1), e, i32) for e in e_list], axis=1)

    iota_er = jax.lax.broadcasted_iota(i32, (NE, 1), 0)
    sel_mat = jnp.concatenate(
        [(iota_er == e).astype(f32) for e in e_list], axis=1)
    cols = jax.lax.dot_general(srw, sel_mat, (((1,), (0,)), ((), ())),
                               preferred_element_type=f32)

    cs_cols = cols[:, TD:2 * TD]
    bv = jax.lax.bitcast_convert_type(cs_cols, i32)
    ri = jax.lax.broadcasted_iota(i32, (bs, TD), 0)
    knn = float(MAXNNZ)

    def t_body(_, lh):
        lo, hi = lh
        mid = lo + (hi - lo + 1) // 2
        cnt = jnp.sum((bv >= mid).astype(f32), axis=0, keepdims=True)
        ge = cnt >= knn
        return jnp.where(ge, mid, lo), jnp.where(ge, hi, mid - 1)

    t0 = (jnp.zeros((1, TD), i32), jnp.full((1, TD), 2**30, i32))
    tthr, _ = jax.lax.fori_loop(0, 31, t_body, t0)

    cgt = jnp.sum((bv > tthr).astype(f32), axis=0, keepdims=True)
    mfill = knn - cgt
    eqm = bv == tthr

    def i_body(_, lh):
        lo, hi = lh
        mid = lo + (hi - lo + 1) // 2
        cnt = jnp.sum((eqm & (ri <= mid)).astype(f32), axis=0, keepdims=True)
        ok = cnt <= mfill
        return jnp.where(ok, mid, lo), jnp.where(ok, hi, mid - 1)

    i0 = (jnp.full((1, TD), -1, i32), jnp.full((1, TD), bs - 1, i32))
    ithr, _ = jax.lax.fori_loop(0, 12, i_body, i0)

    selm = ((bv > tthr) | (eqm & (ri <= ithr))).astype(f32)
    w_ref[...] = jnp.concatenate([cols[:, :TD], cs_cols * selm], axis=1)

    iota_r = jax.lax.broadcasted_iota(i32, (bs, bs), 0)
    iota_c = jax.lax.broadcasted_iota(i32, (bs, bs), 1)
    tri = (iota_c <= iota_r).astype(f32)
    rank_inc = jax.lax.dot_general(tri, selm, (((1,), (0,)), ((), ())),
                                   preferred_element_type=f32)
    slot_ref[...] = jnp.where(selm > 0.0, rank_inc - 1.0, -1.0).astype(i32)


def _mlp_block(x16, u_ref, g_ref, ub_ref, gb_ref, e):
    f32 = jnp.float32
    ublk = u_ref[...].astype(jnp.bfloat16)
    gblk = g_ref[...].astype(jnp.bfloat16)
    ubias = ub_ref[...].reshape(1, EB)
    gbias = gb_ref[...].reshape(1, EB)
    ug = jnp.concatenate([ublk, gblk], axis=0)
    aug = jax.lax.dot_general(x16, ug, (((1,), (1,)), ((), ())),
                              preferred_element_type=f32)
    up = aug[:, :EB] + ubias
    gv = aug[:, EB:] + gbias
    gv = jnp.minimum(gv, LIMIT)
    up = jnp.clip(up, -LIMIT, LIMIT)
    act = gv / (1.0 + jnp.exp(-ALPHA * gv))
    h = (up + 1.0) * act
    col = e * EB + jax.lax.broadcasted_iota(jnp.int32, h.shape, 1)
    return jnp.where(col < OED, h, 0.0)


def _dense_body(mask_ref, x_ref, u_ref, g_ref, d_ref, ub_ref, gb_ref,
                w_ref, out_ref, h_ref):
    c = pl.program_id(0)
    e = pl.program_id(1)
    f32 = jnp.float32

    @pl.when(jnp.logical_and(c == 0, e == 0))
    def _():
        out_ref[...] = jnp.zeros_like(out_ref)

    h = _mlp_block(x_ref[...], u_ref, g_ref, ub_ref, gb_ref, e)

    oh = (jax.lax.broadcasted_iota(jnp.int32, (2 * TD, 1), 0) == c).astype(f32)
    wcol = jax.lax.dot_general(w_ref[...], oh, (((1,), (0,)), ((), ())),
                               preferred_element_type=f32)
    h_ref[:, pl.ds(e * EB, EB)] = (h * wcol).astype(jnp.bfloat16)

    @pl.when(e == NB - 1)
    def _():
        dfull = d_ref[...].astype(jnp.bfloat16)
        out_ref[...] += jax.lax.dot_general(
            h_ref[...], dfull, (((1,), (1,)), ((), ())),
            preferred_element_type=f32)


def _sparse_body(mask_ref, x_ref, slot_ref, u_ref, g_ref, d_ref, ub_ref,
                 gb_ref, w_ref, out_ref, h_ref, p_ref, xs_ref):
    c = pl.program_id(0)
    e = pl.program_id(1)
    f32 = jnp.float32
    bs = x_ref.shape[0]

    @pl.when(jnp.logical_and(c == 0, e == 0))
    def _():
        out_ref[...] = jnp.zeros_like(out_ref)

    @pl.when(e == 0)
    def _():
        srow = slot_ref[...].reshape(1, bs)
        iot = jax.lax.broadcasted_iota(jnp.int32, (MAXNNZ, bs), 0)
        p_ref[...] = (iot == srow).astype(f32)
        xs_ref[...] = jax.lax.dot_general(
            p_ref[...].astype(jnp.bfloat16), x_ref[...],
            (((1,), (0,)), ((), ())),
            preferred_element_type=f32).astype(jnp.bfloat16)

    h = _mlp_block(xs_ref[...], u_ref, g_ref, ub_ref, gb_ref, e)

    oh = (jax.lax.broadcasted_iota(jnp.int32, (2 * TD, 1), 0)
          == c + TD).astype(f32)
    wcol = jax.lax.dot_general(w_ref[...], oh, (((1,), (0,)), ((), ())),
                               preferred_element_type=f32)
    ws = jax.lax.dot_general(p_ref[...], wcol, (((1,), (0,)), ((), ())),
                             preferred_element_type=f32)
    h_ref[:, pl.ds(e * EB, EB)] = (h * ws).astype(jnp.bfloat16)

    @pl.when(e == NB - 1)
    def _():
        dfull = d_ref[...].astype(jnp.bfloat16)
        contrib = jax.lax.dot_general(
            h_ref[...], dfull, (((1,), (1,)), ((), ())),
            preferred_element_type=f32)
        out_ref[...] += jax.lax.dot_general(
            p_ref[...], contrib, (((0,), (0,)), ((), ())),
            preferred_element_type=f32)


def kernel(hid, gate_w, u, g, d, ub, gb):
    b, s, hd = hid.shape
    bs = b * s
    x = hid.reshape(bs, hd)
    ub3 = ub.reshape(NE * NB, 1, EB)
    gb3 = gb.reshape(NE * NB, 1, EB)

    mask_c, w, slot = pl.pallas_call(
        _router_body,
        out_shape=[
            jax.ShapeDtypeStruct((1, 2 * TD), jnp.int32),
            jax.ShapeDtypeStruct((bs, 2 * TD), jnp.float32),
            jax.ShapeDtypeStruct((bs, TD), jnp.int32),
        ],
    )(x, gate_w)

    x16 = x.astype(jnp.bfloat16)
    slot_t = slot.T.reshape(TD, 1, bs)

    dense_spec = pltpu.PrefetchScalarGridSpec(
        num_scalar_prefetch=1,
        grid=(TD, NB),
        in_specs=[
            pl.BlockSpec((bs, hd), lambda c, e, mref: (0, 0)),
            pl.BlockSpec((EB, hd), lambda c, e, mref: (mref[0, c] * NB + e, 0)),
            pl.BlockSpec((EB, hd), lambda c, e, mref: (mref[0, c] * NB + e, 0)),
            pl.BlockSpec((hd, EW), lambda c, e, mref: (0, mref[0, c])),
            pl.BlockSpec((1, 1, EB), lambda c, e, mref: (mref[0, c] * NB + e, 0, 0)),
            pl.BlockSpec((1, 1, EB), lambda c, e, mref: (mref[0, c] * NB + e, 0, 0)),
            pl.BlockSpec((bs, 2 * TD), lambda c, e, mref: (0, 0)),
        ],
        out_specs=pl.BlockSpec((bs, hd), lambda c, e, mref: (0, 0)),
        scratch_shapes=[pltpu.VMEM((bs, EW), jnp.bfloat16)],
    )
    out_d = pl.pallas_call(
        _dense_body,
        grid_spec=dense_spec,
        out_shape=jax.ShapeDtypeStruct((bs, hd), jnp.float32),
        compiler_params=pltpu.CompilerParams(
            dimension_semantics=("arbitrary", "arbitrary")),
    )(mask_c, x16, u, g, d, ub3, gb3, w)

    sparse_spec = pltpu.PrefetchScalarGridSpec(
        num_scalar_prefetch=1,
        grid=(TD, NB),
        in_specs=[
            pl.BlockSpec((bs, hd), lambda c, e, mref: (0, 0)),
            pl.BlockSpec((1, 1, bs), lambda c, e, mref: (c, 0, 0)),
            pl.BlockSpec((EB, hd),
                         lambda c, e, mref: (mref[0, c + TD] * NB + e, 0)),
            pl.BlockSpec((EB, hd),
                         lambda c, e, mref: (mref[0, c + TD] * NB + e, 0)),
            pl.BlockSpec((hd, EW), lambda c, e, mref: (0, mref[0, c + TD])),
            pl.BlockSpec((1, 1, EB),
                         lambda c, e, mref: (mref[0, c + TD] * NB + e, 0, 0)),
            pl.BlockSpec((1, 1, EB),
                         lambda c, e, mref: (mref[0, c + TD] * NB + e, 0, 0)),
            pl.BlockSpec((bs, 2 * TD), lambda c, e, mref: (0, 0)),
        ],
        out_specs=pl.BlockSpec((bs, hd), lambda c, e, mref: (0, 0)),
        scratch_shapes=[
            pltpu.VMEM((MAXNNZ, EW), jnp.bfloat16),
            pltpu.VMEM((MAXNNZ, bs), jnp.float32),
            pltpu.VMEM((MAXNNZ, hd), jnp.bfloat16),
        ],
    )
    out_s = pl.pallas_call(
        _sparse_body,
        grid_spec=sparse_spec,
        out_shape=jax.ShapeDtypeStruct((bs, hd), jnp.float32),
        compiler_params=pltpu.CompilerParams(
            dimension_semantics=("arbitrary", "arbitrary")),
    )(mask_c, x16, slot_t, u, g, d, ub3, gb3, w)
    return out_d + out_s


# speedup vs baseline: 1.2904x; 1.2904x over previous
import jax
import jax.numpy as jnp
from jax.experimental import pallas as pl
from jax.experimental.pallas import tpu as pltpu

HID = 768
NE = 32
EW = 2944
TOPK = 4
TD = 4
MAXNNZ = 512
ALPHA = 1.702
LIMIT = 7.0
OED = 2880

EB = 128
NB = EW // EB


def _router_body(x_ref, gw_ref, mask_ref, w_ref, slot_ref):
    x = x_ref[...]
    gw = gw_ref[...]
    bs = x.shape[0]
    f32 = jnp.float32
    i32 = jnp.int32

    logits = jax.lax.dot_general(x, gw, (((1,), (1,)), ((), ())),
                                 preferred_element_type=f32)
    m = jnp.max(logits, axis=1, keepdims=True)
    ex = jnp.exp(logits - m)
    rw = ex / jnp.sum(ex, axis=1, keepdims=True)

    iota_l = jax.lax.broadcasted_iota(i32, (bs, NE), 1)
    cur = rw
    srw_un = jnp.zeros_like(rw)
    vsum = jnp.zeros((bs, 1), f32)
    for _ in range(TOPK):
        mj = jnp.max(cur, axis=1, keepdims=True)
        ismax = cur == mj
        fidx = jnp.min(jnp.where(ismax, iota_l, NE * 2), axis=1, keepdims=True)
        oh = iota_l == fidx
        srw_un = srw_un + jnp.where(oh, mj, 0.0)
        vsum = vsum + mj
        cur = jnp.where(oh, -1.0, cur)
    srw = srw_un / vsum

    counts = jnp.sum((srw > 0.0).astype(f32), axis=0, keepdims=True)
    iota_e = jax.lax.broadcasted_iota(i32, (1, NE), 1)
    kv = counts.astype(i32) * NE + (NE - 1 - iota_e)
    e_list = []
    for _ in range(2 * TD):
        kmax = jnp.max(kv)
        e_j = jnp.min(jnp.where(kv == kmax, iota_e, NE * 2))
        e_list.append(e_j)
        kv = jnp.where(iota_e == e_j, -1, kv)
    mask_ref[...] = jnp.concatenate(
        [jnp.full((1, 1), e, i32) for e in e_list], axis=1)

    iota_er = jax.lax.broadcasted_iota(i32, (NE, 1), 0)
    sel_mat = jnp.concatenate(
        [(iota_er == e).astype(f32) for e in e_list], axis=1)
    cols = jax.lax.dot_general(srw, sel_mat, (((1,), (0,)), ((), ())),
                               preferred_element_type=f32)

    cs_cols = cols[:, TD:2 * TD]
    bv = jax.lax.bitcast_convert_type(cs_cols, i32)
    ri = jax.lax.broadcasted_iota(i32, (bs, TD), 0)
    knn = float(MAXNNZ)

    def t_body(_, lh):
        lo, hi = lh
        mid = lo + (hi - lo + 1) // 2
        cnt = jnp.sum((bv >= mid).astype(f32), axis=0, keepdims=True)
        ge = cnt >= knn
        return jnp.where(ge, mid, lo), jnp.where(ge, hi, mid - 1)

    t0 = (jnp.zeros((1, TD), i32), jnp.full((1, TD), 2**30, i32))
    tthr, _ = jax.lax.fori_loop(0, 31, t_body, t0)

    cgt = jnp.sum((bv > tthr).astype(f32), axis=0, keepdims=True)
    mfill = knn - cgt
    eqm = bv == tthr

    def i_body(_, lh):
        lo, hi = lh
        mid = lo + (hi - lo + 1) // 2
        cnt = jnp.sum((eqm & (ri <= mid)).astype(f32), axis=0, keepdims=True)
        ok = cnt <= mfill
        return jnp.where(ok, mid, lo), jnp.where(ok, hi, mid - 1)

    i0 = (jnp.full((1, TD), -1, i32), jnp.full((1, TD), bs - 1, i32))
    ithr, _ = jax.lax.fori_loop(0, 12, i_body, i0)

    selm = ((bv > tthr) | (eqm & (ri <= ithr))).astype(f32)
    w_ref[...] = jnp.concatenate([cols[:, :TD], cs_cols * selm], axis=1)

    iota_r = jax.lax.broadcasted_iota(i32, (bs, bs), 0)
    iota_c = jax.lax.broadcasted_iota(i32, (bs, bs), 1)
    tri = (iota_c <= iota_r).astype(f32)
    rank_inc = jax.lax.dot_general(tri, selm, (((1,), (0,)), ((), ())),
                                   preferred_element_type=f32)
    slot_ref[...] = jnp.where(selm > 0.0, rank_inc - 1.0, -1.0).astype(i32)


def _mlp_block(x16, u_ref, g_ref, ub_ref, gb_ref, e):
    f32 = jnp.float32
    ublk = u_ref[...].astype(jnp.bfloat16)
    gblk = g_ref[...].astype(jnp.bfloat16)
    ubias = ub_ref[...].reshape(1, EB)
    gbias = gb_ref[...].reshape(1, EB)
    ug = jnp.concatenate([ublk, gblk], axis=0)
    aug = jax.lax.dot_general(x16, ug, (((1,), (1,)), ((), ())),
                              preferred_element_type=f32)
    up = aug[:, :EB] + ubias
    gv = aug[:, EB:] + gbias
    gv = jnp.minimum(gv, LIMIT)
    up = jnp.clip(up, -LIMIT, LIMIT)
    act = gv / (1.0 + jnp.exp(-ALPHA * gv))
    h = (up + 1.0) * act
    col = e * EB + jax.lax.broadcasted_iota(jnp.int32, h.shape, 1)
    return jnp.where(col < OED, h, 0.0)


def _dense_body(mask_ref, x_ref, u_ref, g_ref, d_ref, ub_ref, gb_ref,
                w_ref, out_ref, h_ref, wc_ref):
    c = pl.program_id(0)
    e = pl.program_id(1)
    f32 = jnp.float32

    @pl.when(jnp.logical_and(c == 0, e == 0))
    def _():
        out_ref[...] = jnp.zeros_like(out_ref)

    @pl.when(e == 0)
    def _():
        oh = (jax.lax.broadcasted_iota(jnp.int32, (2 * TD, 1), 0)
              == c).astype(f32)
        wc_ref[...] = jax.lax.dot_general(
            w_ref[...], oh, (((1,), (0,)), ((), ())),
            preferred_element_type=f32)

    h = _mlp_block(x_ref[...], u_ref, g_ref, ub_ref, gb_ref, e)
    h_ref[:, pl.ds(e * EB, EB)] = (h * wc_ref[...]).astype(jnp.bfloat16)

    @pl.when(e == NB - 1)
    def _():
        dfull = d_ref[...].astype(jnp.bfloat16)
        out_ref[...] += jax.lax.dot_general(
            h_ref[...], dfull, (((1,), (1,)), ((), ())),
            preferred_element_type=f32)


def _sparse_body(mask_ref, x_ref, slot_ref, u_ref, g_ref, d_ref, ub_ref,
                 gb_ref, w_ref, out_ref, h_ref, p_ref, xs_ref, ws_ref):
    c = pl.program_id(0)
    e = pl.program_id(1)
    f32 = jnp.float32
    bs = x_ref.shape[0]

    @pl.when(jnp.logical_and(c == 0, e == 0))
    def _():
        out_ref[...] = jnp.zeros_like(out_ref)

    @pl.when(e == 0)
    def _():
        srow = slot_ref[...].reshape(1, bs)
        iot = jax.lax.broadcasted_iota(jnp.int32, (MAXNNZ, bs), 0)
        p_ref[...] = (iot == srow).astype(f32)
        xs_ref[...] = jax.lax.dot_general(
            p_ref[...].astype(jnp.bfloat16), x_ref[...],
            (((1,), (0,)), ((), ())),
            preferred_element_type=f32).astype(jnp.bfloat16)
        oh = (jax.lax.broadcasted_iota(jnp.int32, (2 * TD, 1), 0)
              == c + TD).astype(f32)
        wcol = jax.lax.dot_general(w_ref[...], oh, (((1,), (0,)), ((), ())),
                                   preferred_element_type=f32)
        ws_ref[...] = jax.lax.dot_general(
            p_ref[...], wcol, (((1,), (0,)), ((), ())),
            preferred_element_type=f32)

    h = _mlp_block(xs_ref[...], u_ref, g_ref, ub_ref, gb_ref, e)
    h_ref[:, pl.ds(e * EB, EB)] = (h * ws_ref[...]).astype(jnp.bfloat16)

    @pl.when(e == NB - 1)
    def _():
        dfull = d_ref[...].astype(jnp.bfloat16)
        contrib = jax.lax.dot_general(
            h_ref[...], dfull, (((1,), (1,)), ((), ())),
            preferred_element_type=f32)
        out_ref[...] += jax.lax.dot_general(
            p_ref[...], contrib, (((0,), (0,)), ((), ())),
            preferred_element_type=f32)


def kernel(hid, gate_w, u, g, d, ub, gb):
    b, s, hd = hid.shape
    bs = b * s
    x = hid.reshape(bs, hd)
    ub3 = ub.reshape(NE * NB, 1, EB)
    gb3 = gb.reshape(NE * NB, 1, EB)

    mask_c, w, slot = pl.pallas_call(
        _router_body,
        out_shape=[
            jax.ShapeDtypeStruct((1, 2 * TD), jnp.int32),
            jax.ShapeDtypeStruct((bs, 2 * TD), jnp.float32),
            jax.ShapeDtypeStruct((bs, TD), jnp.int32),
        ],
    )(x, gate_w)

    x16 = x.astype(jnp.bfloat16)
    slot_t = slot.T.reshape(TD, 1, bs)

    dense_spec = pltpu.PrefetchScalarGridSpec(
        num_scalar_prefetch=1,
        grid=(TD, NB),
        in_specs=[
            pl.BlockSpec((bs, hd), lambda c, e, mref: (0, 0)),
            pl.BlockSpec((EB, hd), lambda c, e, mref: (mref[0, c] * NB + e, 0)),
            pl.BlockSpec((EB, hd), lambda c, e, mref: (mref[0, c] * NB + e, 0)),
            pl.BlockSpec((hd, EW), lambda c, e, mref: (0, mref[0, c])),
            pl.BlockSpec((1, 1, EB), lambda c, e, mref: (mref[0, c] * NB + e, 0, 0)),
            pl.BlockSpec((1, 1, EB), lambda c, e, mref: (mref[0, c] * NB + e, 0, 0)),
            pl.BlockSpec((bs, 2 * TD), lambda c, e, mref: (0, 0)),
        ],
        out_specs=pl.BlockSpec((bs, hd), lambda c, e, mref: (0, 0)),
        scratch_shapes=[pltpu.VMEM((bs, EW), jnp.bfloat16),
                        pltpu.VMEM((bs, 1), jnp.float32)],
    )
    out_d = pl.pallas_call(
        _dense_body,
        grid_spec=dense_spec,
        out_shape=jax.ShapeDtypeStruct((bs, hd), jnp.float32),
        compiler_params=pltpu.CompilerParams(
            dimension_semantics=("arbitrary", "arbitrary")),
    )(mask_c, x16, u, g, d, ub3, gb3, w)

    sparse_spec = pltpu.PrefetchScalarGridSpec(
        num_scalar_prefetch=1,
        grid=(TD, NB),
        in_specs=[
            pl.BlockSpec((bs, hd), lambda c, e, mref: (0, 0)),
            pl.BlockSpec((1, 1, bs), lambda c, e, mref: (c, 0, 0)),
            pl.BlockSpec((EB, hd),
                         lambda c, e, mref: (mref[0, c + TD] * NB + e, 0)),
            pl.BlockSpec((EB, hd),
                         lambda c, e, mref: (mref[0, c + TD] * NB + e, 0)),
            pl.BlockSpec((hd, EW), lambda c, e, mref: (0, mref[0, c + TD])),
            pl.BlockSpec((1, 1, EB),
                         lambda c, e, mref: (mref[0, c + TD] * NB + e, 0, 0)),
            pl.BlockSpec((1, 1, EB),
                         lambda c, e, mref: (mref[0, c + TD] * NB + e, 0, 0)),
            pl.BlockSpec((bs, 2 * TD), lambda c, e, mref: (0, 0)),
        ],
        out_specs=pl.BlockSpec((bs, hd), lambda c, e, mref: (0, 0)),
        scratch_shapes=[
            pltpu.VMEM((MAXNNZ, EW), jnp.bfloat16),
            pltpu.VMEM((MAXNNZ, bs), jnp.float32),
            pltpu.VMEM((MAXNNZ, hd), jnp.bfloat16),
            pltpu.VMEM((MAXNNZ, 1), jnp.float32),
        ],
    )
    out_s = pl.pallas_call(
        _sparse_body,
        grid_spec=sparse_spec,
        out_shape=jax.ShapeDtypeStruct((bs, hd), jnp.float32),
        compiler_params=pltpu.CompilerParams(
            dimension_semantics=("arbitrary", "arbitrary")),
    )(mask_c, x16, slot_t, u, g, d, ub3, gb3, w)
    return out_d + out_s
